# baseline (speedup 1.0000x reference)
import functools

import jax
import jax.numpy as jnp
from jax import lax
from jax.experimental import pallas as pl
from jax.experimental.pallas import tpu as pltpu
from jax.experimental.pallas import tpu_sc as plsc

F32 = jnp.float32
NC, NS = 2, 16
NW = NC * NS
CHUNK = 128
LN_EPS = 1e-5


def _sc_mesh():
    return plsc.VectorSubcoreMesh(core_axis_name="c", subcore_axis_name="s")


def _tc_precompute(x, w_s, w_d):
    n, d = x.shape
    blk = 2000
    h = w_s.shape[1]

    def body(x_ref, ws_ref, wd_ref, xs_ref, xd_ref):
        xb = x_ref[...]
        xs_ref[...] = jnp.dot(xb, ws_ref[...], preferred_element_type=F32)
        xd_ref[...] = jnp.dot(xb, wd_ref[...], preferred_element_type=F32)

    return pl.pallas_call(
        body,
        grid=(n // blk,),
        in_specs=[
            pl.BlockSpec((blk, d), lambda i: (i, 0)),
            pl.BlockSpec((d, h), lambda i: (0, 0)),
            pl.BlockSpec((d, h), lambda i: (0, 0)),
        ],
        out_specs=[
            pl.BlockSpec((blk, h), lambda i: (i, 0)),
            pl.BlockSpec((blk, h), lambda i: (i, 0)),
        ],
        out_shape=[
            jax.ShapeDtypeStruct((n, h), F32),
            jax.ShapeDtypeStruct((n, h), F32),
        ],
    )(x, w_s, w_d)


def _sc_gather(xs, xd, rowg, colg, zeros1, ones1, epad, npad):
    h = xs.shape[1]
    per_w = epad // NW
    nchunks = per_w // CHUNK
    rows_per_tile = npad // NS
    spans = _row_spans(rows_per_tile)

    @functools.partial(
        pl.kernel,
        out_type=(
            jax.ShapeDtypeStruct((epad, h), F32),
            jax.ShapeDtypeStruct((epad, h), F32),
            jax.ShapeDtypeStruct((NC * npad,), F32),
        ),
        mesh=_sc_mesh(),
        scratch_types=[
            pltpu.VMEM((CHUNK,), jnp.int32),
            pltpu.VMEM((CHUNK,), jnp.int32),
            pltpu.VMEM((CHUNK,), jnp.int32),
            pltpu.VMEM((CHUNK,), jnp.int32),
            pltpu.VMEM((CHUNK, h), F32),
            pltpu.VMEM((CHUNK, h), F32),
            pltpu.VMEM((CHUNK, h), F32),
            pltpu.VMEM((CHUNK, h), F32),
            pltpu.VMEM((CHUNK,), F32),
            pltpu.VMEM((CHUNK,), F32),
            pltpu.VMEM_SHARED((npad,), F32),
        ] + [pltpu.SemaphoreType.DMA] * 14,
    )
    def k(xs_hbm, xd_hbm, row_hbm, col_hbm, z_hbm, ones_hbm,
          gs_hbm, gd_hbm, cnt_hbm,
          ia0, ib0, ia1, ib1, ra0, rb0, ra1, rb1, stage_v, ones_v, sh_cnt,
          s0, s1, s2, s3, s4, s5, s6, s7, s8, s9, s10, s11, s12, s13):
        cid = lax.axis_index("c")
        sid = lax.axis_index("s")
        wid = sid * NC + cid
        base_w = wid * per_w

        pltpu.sync_copy(z_hbm, stage_v)
        for joff, jlen in spans:
            zbase = sid * rows_per_tile + joff
            pltpu.sync_copy(stage_v.at[pl.ds(0, jlen)],
                            sh_cnt.at[pl.ds(zbase, jlen)])
        pltpu.sync_copy(ones_hbm, ones_v)
        plsc.subcore_barrier()

        def body(j, carry):
            b0 = base_w + (2 * j) * CHUNK
            b1 = b0 + CHUNK
            c1 = pltpu.async_copy(row_hbm.at[pl.ds(b0, CHUNK)], ia0, s0)
            c2 = pltpu.async_copy(col_hbm.at[pl.ds(b0, CHUNK)], ib0, s1)
            c3 = pltpu.async_copy(row_hbm.at[pl.ds(b1, CHUNK)], ia1, s2)
            c4 = pltpu.async_copy(col_hbm.at[pl.ds(b1, CHUNK)], ib1, s3)
            c1.wait()
            g1 = pltpu.async_copy(xs_hbm.at[ia0], ra0, s4)
            c2.wait()
            g2 = pltpu.async_copy(xd_hbm.at[ib0], rb0, s5)
            a0 = pltpu.async_copy(ones_v, sh_cnt.at[ib0], s12, add=True)
            c3.wait()
            g3 = pltpu.async_copy(xs_hbm.at[ia1], ra1, s6)
            c4.wait()
            g4 = pltpu.async_copy(xd_hbm.at[ib1], rb1, s7)
            a1 = pltpu.async_copy(ones_v, sh_cnt.at[ib1], s13, add=True)
            g1.wait()
            w1 = pltpu.async_copy(ra0, gs_hbm.at[pl.ds(b0, CHUNK)], s8)
            g2.wait()
            w2 = pltpu.async_copy(rb0, gd_hbm.at[pl.ds(b0, CHUNK)], s9)
            g3.wait()
            w3 = pltpu.async_copy(ra1, gs_hbm.at[pl.ds(b1, CHUNK)], s10)
            g4.wait()
            w4 = pltpu.async_copy(rb1, gd_hbm.at[pl.ds(b1, CHUNK)], s11)
            a0.wait()
            a1.wait()
            w1.wait()
            w2.wait()
            w3.wait()
            w4.wait()
            return carry

        lax.fori_loop(0, nchunks // 2, body, 0)
        plsc.subcore_barrier()

        def copy_out(c):
            for joff, jlen in spans:
                obase = sid * rows_per_tile + joff
                pltpu.sync_copy(sh_cnt.at[pl.ds(obase, jlen)],
                                stage_v.at[pl.ds(0, jlen)])
                pltpu.sync_copy(stage_v.at[pl.ds(0, jlen)],
                                cnt_hbm.at[pl.ds(c * npad + obase, jlen)])

        @pl.when(cid == 0)
        def _():
            copy_out(0)

        @pl.when(cid == 1)
        def _():
            copy_out(1)

    return k(xs, xd, rowg, colg, zeros1, ones1)


def _tc_edge(gs, gd, ea, w1e, b1, g, b, w2, b2):
    epad, h = gs.shape
    ed = ea.shape[1]
    blk = 2048

    def body(gs_ref, gd_ref, ea_ref, w1e_ref, b1_ref, g_ref, b_ref,
             w2_ref, b2_ref, out_ref):
        pre = (gs_ref[...] + gd_ref[...]
               + jnp.dot(ea_ref[...], w1e_ref[...], preferred_element_type=F32)
               + b1_ref[...])
        hh = jnp.maximum(pre, 0.0)
        m = jnp.mean(hh, axis=-1, keepdims=True)
        c = hh - m
        v = jnp.mean(c * c, axis=-1, keepdims=True)
        hn = c * lax.rsqrt(v + LN_EPS) * g_ref[...] + b_ref[...]
        out_ref[...] = (jnp.dot(hn, w2_ref[...], preferred_element_type=F32)
                        + b2_ref[...])

    return pl.pallas_call(
        body,
        grid=(epad // blk,),
        in_specs=[
            pl.BlockSpec((blk, h), lambda i: (i, 0)),
            pl.BlockSpec((blk, h), lambda i: (i, 0)),
            pl.BlockSpec((blk, ed), lambda i: (i, 0)),
            pl.BlockSpec((ed, h), lambda i: (0, 0)),
            pl.BlockSpec((1, h), lambda i: (0, 0)),
            pl.BlockSpec((1, h), lambda i: (0, 0)),
            pl.BlockSpec((1, h), lambda i: (0, 0)),
            pl.BlockSpec((h, h), lambda i: (0, 0)),
            pl.BlockSpec((1, h), lambda i: (0, 0)),
        ],
        out_specs=pl.BlockSpec((blk, h), lambda i: (i, 0)),
        out_shape=jax.ShapeDtypeStruct((epad, h), F32),
    )(gs, gd, ea, w1e, b1, g, b, w2, b2)


def _row_spans(rows_per_tile):
    spans = []
    off = 0
    while off < rows_per_tile:
        c = min(CHUNK, rows_per_tile - off)
        spans.append((off, c))
        off += c
    return spans


def _sc_scatter_agg(ean, cols, zeros_agg, npad):
    epad, h = ean.shape
    per_w = epad // NW
    nchunks = per_w // CHUNK
    rows_per_tile = npad // NS
    spans = _row_spans(rows_per_tile)

    @functools.partial(
        pl.kernel,
        out_type=jax.ShapeDtypeStruct((NC, npad, h), F32),
        mesh=_sc_mesh(),
        scratch_types=[
            pltpu.VMEM((CHUNK,), jnp.int32),
            pltpu.VMEM((CHUNK,), jnp.int32),
            pltpu.VMEM((CHUNK, h), F32),
            pltpu.VMEM((CHUNK, h), F32),
            pltpu.VMEM_SHARED((npad, h), F32),
        ] + [pltpu.SemaphoreType.DMA] * 6,
    )
    def k(ean_hbm, col_hbm, za_hbm, agg_hbm, idx0, idx1, rows0, rows1,
          sh_agg, s0, s1, s2, s3, s4, s5):
        cid = lax.axis_index("c")
        sid = lax.axis_index("s")
        wid = sid * NC + cid

        pltpu.sync_copy(za_hbm, rows0)
        for joff, jlen in spans:
            zbase = sid * rows_per_tile + joff
            pltpu.sync_copy(rows0.at[pl.ds(0, jlen)],
                            sh_agg.at[pl.ds(zbase, jlen)])
        plsc.subcore_barrier()

        base_w = wid * per_w

        def body(j, carry):
            b0 = base_w + (2 * j) * CHUNK
            b1 = b0 + CHUNK
            c0 = pltpu.async_copy(col_hbm.at[pl.ds(b0, CHUNK)], idx0, s0)
            c1 = pltpu.async_copy(col_hbm.at[pl.ds(b1, CHUNK)], idx1, s1)
            r0 = pltpu.async_copy(ean_hbm.at[pl.ds(b0, CHUNK)], rows0, s2)
            r1 = pltpu.async_copy(ean_hbm.at[pl.ds(b1, CHUNK)], rows1, s3)
            c0.wait()
            r0.wait()
            a0 = pltpu.async_copy(rows0, sh_agg.at[idx0], s4, add=True)
            c1.wait()
            r1.wait()
            a1 = pltpu.async_copy(rows1, sh_agg.at[idx1], s5, add=True)
            a0.wait()
            a1.wait()
            return carry

        lax.fori_loop(0, nchunks // 2, body, 0)
        plsc.subcore_barrier()

        def copy_out(c):
            for joff, jlen in spans:
                obase = sid * rows_per_tile + joff
                pltpu.sync_copy(sh_agg.at[pl.ds(obase, jlen)],
                                rows0.at[pl.ds(0, jlen)])
                pltpu.sync_copy(rows0.at[pl.ds(0, jlen)],
                                agg_hbm.at[c, pl.ds(obase, jlen)])

        @pl.when(cid == 0)
        def _():
            copy_out(0)

        @pl.when(cid == 1)
        def _():
            copy_out(1)

    return k(ean, cols, zeros_agg)


def _tc_node(x, agg_p, c0, c1, w1x, w1a, b1, g, b, w2, b2):
    n, d = x.shape
    h = w1x.shape[1]
    cw = c0.shape[1]
    blk = 2000

    def body(x_ref, a_ref, c0_ref, c1_ref, w1x_ref, w1a_ref,
             b1_ref, g_ref, b_ref, w2_ref, b2_ref, out_ref):
        cnt = c0_ref[...][:, :1] + c1_ref[...][:, :1]
        inv = 1.0 / jnp.maximum(cnt, 1.0)
        agg = (a_ref[0] + a_ref[1]) * inv
        gg = (jnp.dot(x_ref[...], w1x_ref[...], preferred_element_type=F32)
              + jnp.dot(agg, w1a_ref[...], preferred_element_type=F32)
              + b1_ref[...])
        hh_ = jnp.maximum(gg, 0.0)
        m = jnp.mean(hh_, axis=-1, keepdims=True)
        c = hh_ - m
        v = jnp.mean(c * c, axis=-1, keepdims=True)
        hn = c * lax.rsqrt(v + LN_EPS) * g_ref[...] + b_ref[...]
        out_ref[...] = (jnp.dot(hn, w2_ref[...], preferred_element_type=F32)
                        + b2_ref[...])

    return pl.pallas_call(
        body,
        grid=(n // blk,),
        in_specs=[
            pl.BlockSpec((blk, d), lambda i: (i, 0)),
            pl.BlockSpec((NC, blk, h), lambda i: (0, i, 0)),
            pl.BlockSpec((blk, cw), lambda i: (i, 0)),
            pl.BlockSpec((blk, cw), lambda i: (i, 0)),
            pl.BlockSpec((d, h), lambda i: (0, 0)),
            pl.BlockSpec((h, h), lambda i: (0, 0)),
            pl.BlockSpec((1, h), lambda i: (0, 0)),
            pl.BlockSpec((1, h), lambda i: (0, 0)),
            pl.BlockSpec((1, h), lambda i: (0, 0)),
            pl.BlockSpec((h, h), lambda i: (0, 0)),
            pl.BlockSpec((1, h), lambda i: (0, 0)),
        ],
        out_specs=pl.BlockSpec((blk, h), lambda i: (i, 0)),
        out_shape=jax.ShapeDtypeStruct((n, h), F32),
    )(x, agg_p, c0, c1, w1x, w1a, b1, g, b, w2, b2)


def kernel(x, edge_index, edge_attr, e_W1, e_b1, e_ln_g, e_ln_b, e_W2, e_b2,
           n_W1, n_b1, n_ln_g, n_ln_b, n_W2, n_b2):
    n, d = x.shape
    e, ed = edge_attr.shape
    h = e_W2.shape[1]

    per_w_chunks = -(-e // (NW * CHUNK))
    per_w_chunks += per_w_chunks % 2
    epad = NW * CHUNK * per_w_chunks
    pad = epad - e

    row = edge_index[0]
    col = edge_index[1]
    pad_g = jnp.arange(pad, dtype=jnp.int32) % jnp.int32(128)
    rowg = jnp.concatenate([row, pad_g])
    pad_s = jnp.int32(n) + jnp.arange(pad, dtype=jnp.int32) % jnp.int32(16)
    cols = jnp.concatenate([col, pad_s])
    ea_pad = jnp.concatenate([edge_attr, jnp.zeros((pad, ed), F32)], axis=0)

    w1s = e_W1[:d]
    w1d = e_W1[d:2 * d]
    w1e = e_W1[2 * d:]

    npad = -(-(n + 16) // 128) * 128

    xs, xd = _tc_precompute(x, w1s, w1d)
    xd_pad = jnp.concatenate([xd, jnp.zeros((16, h), F32)], axis=0)
    gs, gd, cnt1d = _sc_gather(xs, xd_pad, rowg, cols,
                               jnp.zeros((CHUNK,), F32),
                               jnp.ones((CHUNK,), F32), epad, npad)
    ean_pad = _tc_edge(gs, gd, ea_pad, w1e,
                       e_b1.reshape(1, -1), e_ln_g.reshape(1, -1),
                       e_ln_b.reshape(1, -1), e_W2, e_b2.reshape(1, -1))
    edge_attr_new = ean_pad[:e]

    agg_p = _sc_scatter_agg(ean_pad, cols, jnp.zeros((CHUNK, h), F32), npad)
    c0 = cnt1d[:npad].reshape(npad, 1)
    c1 = cnt1d[npad:].reshape(npad, 1)

    x_new = _tc_node(x, agg_p, c0, c1,
                     n_W1[:d], n_W1[d:],
                     n_b1.reshape(1, -1), n_ln_g.reshape(1, -1),
                     n_ln_b.reshape(1, -1), n_W2, n_b2.reshape(1, -1))
    return (x_new, edge_attr_new)


# speedup vs baseline: 1.0370x; 1.0370x over previous
import functools

import jax
import jax.numpy as jnp
from jax import lax
from jax.experimental import pallas as pl
from jax.experimental.pallas import tpu as pltpu
from jax.experimental.pallas import tpu_sc as plsc

F32 = jnp.float32
NC, NS = 2, 16
NW = NC * NS
CHUNK = 128
LN_EPS = 1e-5


def _sc_mesh():
    return plsc.VectorSubcoreMesh(core_axis_name="c", subcore_axis_name="s")


def _tc_precompute(x, w_s, w_d):
    n, d = x.shape
    blk = 2000
    h = w_s.shape[1]

    def body(x_ref, ws_ref, wd_ref, xs_ref, xd_ref):
        xb = x_ref[...]
        xs_ref[...] = jnp.dot(xb, ws_ref[...], preferred_element_type=F32)
        xd_ref[...] = jnp.dot(xb, wd_ref[...], preferred_element_type=F32)

    return pl.pallas_call(
        body,
        grid=(n // blk,),
        in_specs=[
            pl.BlockSpec((blk, d), lambda i: (i, 0)),
            pl.BlockSpec((d, h), lambda i: (0, 0)),
            pl.BlockSpec((d, h), lambda i: (0, 0)),
        ],
        out_specs=[
            pl.BlockSpec((blk, h), lambda i: (i, 0)),
            pl.BlockSpec((blk, h), lambda i: (i, 0)),
        ],
        out_shape=[
            jax.ShapeDtypeStruct((n, h), F32),
            jax.ShapeDtypeStruct((n, h), F32),
        ],
    )(x, w_s, w_d)


def _sc_gather(xs, xd, rowg, colg, epad):
    h = xs.shape[1]
    per_w = epad // NW
    nchunks = per_w // CHUNK

    @functools.partial(
        pl.kernel,
        out_type=(
            jax.ShapeDtypeStruct((epad, h), F32),
            jax.ShapeDtypeStruct((epad, h), F32),
        ),
        mesh=_sc_mesh(),
        scratch_types=[
            pltpu.VMEM((CHUNK,), jnp.int32),
            pltpu.VMEM((CHUNK,), jnp.int32),
            pltpu.VMEM((CHUNK,), jnp.int32),
            pltpu.VMEM((CHUNK,), jnp.int32),
            pltpu.VMEM((CHUNK, h), F32),
            pltpu.VMEM((CHUNK, h), F32),
            pltpu.VMEM((CHUNK, h), F32),
            pltpu.VMEM((CHUNK, h), F32),
        ] + [pltpu.SemaphoreType.DMA] * 12,
    )
    def k(xs_hbm, xd_hbm, row_hbm, col_hbm, gs_hbm, gd_hbm,
          ia0, ib0, ia1, ib1, ra0, rb0, ra1, rb1,
          s0, s1, s2, s3, s4, s5, s6, s7, s8, s9, s10, s11):
        wid = lax.axis_index("s") * NC + lax.axis_index("c")
        base_w = wid * per_w

        def drain_writes(b0, b1):
            pltpu.make_async_copy(ra0, gs_hbm.at[pl.ds(b0, CHUNK)], s8).wait()
            pltpu.make_async_copy(rb0, gd_hbm.at[pl.ds(b0, CHUNK)], s9).wait()
            pltpu.make_async_copy(ra1, gs_hbm.at[pl.ds(b1, CHUNK)], s10).wait()
            pltpu.make_async_copy(rb1, gd_hbm.at[pl.ds(b1, CHUNK)], s11).wait()

        def body(j, carry):
            b0 = base_w + (2 * j) * CHUNK
            b1 = b0 + CHUNK
            c1 = pltpu.async_copy(row_hbm.at[pl.ds(b0, CHUNK)], ia0, s0)
            c2 = pltpu.async_copy(col_hbm.at[pl.ds(b0, CHUNK)], ib0, s1)
            c3 = pltpu.async_copy(row_hbm.at[pl.ds(b1, CHUNK)], ia1, s2)
            c4 = pltpu.async_copy(col_hbm.at[pl.ds(b1, CHUNK)], ib1, s3)

            @pl.when(j > 0)
            def _():
                drain_writes(b0, b1)

            c1.wait()
            g1 = pltpu.async_copy(xs_hbm.at[ia0], ra0, s4)
            c2.wait()
            g2 = pltpu.async_copy(xd_hbm.at[ib0], rb0, s5)
            c3.wait()
            g3 = pltpu.async_copy(xs_hbm.at[ia1], ra1, s6)
            c4.wait()
            g4 = pltpu.async_copy(xd_hbm.at[ib1], rb1, s7)
            g1.wait()
            pltpu.async_copy(ra0, gs_hbm.at[pl.ds(b0, CHUNK)], s8)
            g2.wait()
            pltpu.async_copy(rb0, gd_hbm.at[pl.ds(b0, CHUNK)], s9)
            g3.wait()
            pltpu.async_copy(ra1, gs_hbm.at[pl.ds(b1, CHUNK)], s10)
            g4.wait()
            pltpu.async_copy(rb1, gd_hbm.at[pl.ds(b1, CHUNK)], s11)
            return carry

        lax.fori_loop(0, nchunks // 2, body, 0)
        drain_writes(base_w, base_w + CHUNK)

    return k(xs, xd, rowg, colg)


def _tc_edge(gs, gd, ea, w1e, b1, g, b, w2, b2):
    epad, h = gs.shape
    ed = ea.shape[1]
    blk = 2048

    def body(gs_ref, gd_ref, ea_ref, w1e_ref, b1_ref, g_ref, b_ref,
             w2_ref, b2_ref, out_ref):
        pre = (gs_ref[...] + gd_ref[...]
               + jnp.dot(ea_ref[...], w1e_ref[...], preferred_element_type=F32)
               + b1_ref[...])
        hh = jnp.maximum(pre, 0.0)
        m = jnp.mean(hh, axis=-1, keepdims=True)
        c = hh - m
        v = jnp.mean(c * c, axis=-1, keepdims=True)
        hn = c * lax.rsqrt(v + LN_EPS) * g_ref[...] + b_ref[...]
        out_ref[...] = (jnp.dot(hn, w2_ref[...], preferred_element_type=F32)
                        + b2_ref[...])

    return pl.pallas_call(
        body,
        grid=(epad // blk,),
        in_specs=[
            pl.BlockSpec((blk, h), lambda i: (i, 0)),
            pl.BlockSpec((blk, h), lambda i: (i, 0)),
            pl.BlockSpec((blk, ed), lambda i: (i, 0)),
            pl.BlockSpec((ed, h), lambda i: (0, 0)),
            pl.BlockSpec((1, h), lambda i: (0, 0)),
            pl.BlockSpec((1, h), lambda i: (0, 0)),
            pl.BlockSpec((1, h), lambda i: (0, 0)),
            pl.BlockSpec((h, h), lambda i: (0, 0)),
            pl.BlockSpec((1, h), lambda i: (0, 0)),
        ],
        out_specs=pl.BlockSpec((blk, h), lambda i: (i, 0)),
        out_shape=jax.ShapeDtypeStruct((epad, h), F32),
    )(gs, gd, ea, w1e, b1, g, b, w2, b2)


def _row_spans(rows_per_tile):
    spans = []
    off = 0
    while off < rows_per_tile:
        c = min(CHUNK, rows_per_tile - off)
        spans.append((off, c))
        off += c
    return spans


def _sc_scatter_agg(ean, cols, zeros_agg, npad):
    epad, h = ean.shape
    per_w = epad // NW
    nchunks = per_w // CHUNK
    rows_per_tile = npad // NS
    spans = _row_spans(rows_per_tile)

    @functools.partial(
        pl.kernel,
        out_type=jax.ShapeDtypeStruct((NC, npad, h), F32),
        mesh=_sc_mesh(),
        scratch_types=[
            pltpu.VMEM((CHUNK,), jnp.int32),
            pltpu.VMEM((CHUNK,), jnp.int32),
            pltpu.VMEM((CHUNK, h), F32),
            pltpu.VMEM((CHUNK, h), F32),
            pltpu.VMEM_SHARED((npad, h), F32),
        ] + [pltpu.SemaphoreType.DMA] * 6,
    )
    def k(ean_hbm, col_hbm, za_hbm, agg_hbm, idx0, idx1, rows0, rows1,
          sh_agg, s0, s1, s2, s3, s4, s5):
        cid = lax.axis_index("c")
        sid = lax.axis_index("s")
        wid = sid * NC + cid

        pltpu.sync_copy(za_hbm, rows0)
        for joff, jlen in spans:
            zbase = sid * rows_per_tile + joff
            pltpu.sync_copy(rows0.at[pl.ds(0, jlen)],
                            sh_agg.at[pl.ds(zbase, jlen)])
        plsc.subcore_barrier()

        base_w = wid * per_w

        def body(j, carry):
            b0 = base_w + (2 * j) * CHUNK
            b1 = b0 + CHUNK
            c0 = pltpu.async_copy(col_hbm.at[pl.ds(b0, CHUNK)], idx0, s0)
            c1 = pltpu.async_copy(col_hbm.at[pl.ds(b1, CHUNK)], idx1, s1)
            r0 = pltpu.async_copy(ean_hbm.at[pl.ds(b0, CHUNK)], rows0, s2)
            r1 = pltpu.async_copy(ean_hbm.at[pl.ds(b1, CHUNK)], rows1, s3)
            c0.wait()
            r0.wait()
            a0 = pltpu.async_copy(rows0, sh_agg.at[idx0], s4, add=True)
            c1.wait()
            r1.wait()
            a1 = pltpu.async_copy(rows1, sh_agg.at[idx1], s5, add=True)
            a0.wait()
            a1.wait()
            return carry

        lax.fori_loop(0, nchunks // 2, body, 0)
        plsc.subcore_barrier()

        def copy_out(c):
            for joff, jlen in spans:
                obase = sid * rows_per_tile + joff
                pltpu.sync_copy(sh_agg.at[pl.ds(obase, jlen)],
                                rows0.at[pl.ds(0, jlen)])
                pltpu.sync_copy(rows0.at[pl.ds(0, jlen)],
                                agg_hbm.at[c, pl.ds(obase, jlen)])

        @pl.when(cid == 0)
        def _():
            copy_out(0)

        @pl.when(cid == 1)
        def _():
            copy_out(1)

    return k(ean, cols, zeros_agg)


def _sc_scatter_cnt(cols, zeros1, ones1, epad, npad):
    per_w = epad // NW
    nchunks = per_w // CHUNK
    rows_per_tile = npad // NS
    spans = _row_spans(rows_per_tile)

    @functools.partial(
        pl.kernel,
        out_type=jax.ShapeDtypeStruct((NC * npad,), F32),
        mesh=_sc_mesh(),
        scratch_types=[
            pltpu.VMEM((CHUNK,), jnp.int32),
            pltpu.VMEM((CHUNK,), F32),
            pltpu.VMEM((CHUNK,), F32),
            pltpu.VMEM_SHARED((npad,), F32),
        ],
    )
    def k(col_hbm, z_hbm, ones_hbm, cnt_hbm, idx_v, stage_v, ones_v, sh_cnt):
        cid = lax.axis_index("c")
        sid = lax.axis_index("s")
        wid = sid * NC + cid

        pltpu.sync_copy(z_hbm, stage_v)
        for joff, jlen in spans:
            zbase = sid * rows_per_tile + joff
            pltpu.sync_copy(stage_v.at[pl.ds(0, jlen)],
                            sh_cnt.at[pl.ds(zbase, jlen)])
        pltpu.sync_copy(ones_hbm, ones_v)
        plsc.subcore_barrier()

        base_w = wid * per_w

        def body(i, carry):
            base = base_w + i * CHUNK
            pltpu.sync_copy(col_hbm.at[pl.ds(base, CHUNK)], idx_v)
            pltpu.sync_copy(ones_v, sh_cnt.at[idx_v], add=True)
            return carry

        lax.fori_loop(0, nchunks, body, 0)
        plsc.subcore_barrier()

        def copy_out(c):
            for joff, jlen in spans:
                obase = sid * rows_per_tile + joff
                pltpu.sync_copy(sh_cnt.at[pl.ds(obase, jlen)],
                                stage_v.at[pl.ds(0, jlen)])
                pltpu.sync_copy(stage_v.at[pl.ds(0, jlen)],
                                cnt_hbm.at[pl.ds(c * npad + obase, jlen)])

        @pl.when(cid == 0)
        def _():
            copy_out(0)

        @pl.when(cid == 1)
        def _():
            copy_out(1)

    return k(cols, zeros1, ones1)


def _tc_node(x, agg_p, c0, c1, w1x, w1a, b1, g, b, w2, b2):
    n, d = x.shape
    h = w1x.shape[1]
    cw = c0.shape[1]
    blk = 2000

    def body(x_ref, a_ref, c0_ref, c1_ref, w1x_ref, w1a_ref,
             b1_ref, g_ref, b_ref, w2_ref, b2_ref, out_ref):
        cnt = c0_ref[...][:, :1] + c1_ref[...][:, :1]
        inv = 1.0 / jnp.maximum(cnt, 1.0)
        agg = (a_ref[0] + a_ref[1]) * inv
        gg = (jnp.dot(x_ref[...], w1x_ref[...], preferred_element_type=F32)
              + jnp.dot(agg, w1a_ref[...], preferred_element_type=F32)
              + b1_ref[...])
        hh_ = jnp.maximum(gg, 0.0)
        m = jnp.mean(hh_, axis=-1, keepdims=True)
        c = hh_ - m
        v = jnp.mean(c * c, axis=-1, keepdims=True)
        hn = c * lax.rsqrt(v + LN_EPS) * g_ref[...] + b_ref[...]
        out_ref[...] = (jnp.dot(hn, w2_ref[...], preferred_element_type=F32)
                        + b2_ref[...])

    return pl.pallas_call(
        body,
        grid=(n // blk,),
        in_specs=[
            pl.BlockSpec((blk, d), lambda i: (i, 0)),
            pl.BlockSpec((NC, blk, h), lambda i: (0, i, 0)),
            pl.BlockSpec((blk, cw), lambda i: (i, 0)),
            pl.BlockSpec((blk, cw), lambda i: (i, 0)),
            pl.BlockSpec((d, h), lambda i: (0, 0)),
            pl.BlockSpec((h, h), lambda i: (0, 0)),
            pl.BlockSpec((1, h), lambda i: (0, 0)),
            pl.BlockSpec((1, h), lambda i: (0, 0)),
            pl.BlockSpec((1, h), lambda i: (0, 0)),
            pl.BlockSpec((h, h), lambda i: (0, 0)),
            pl.BlockSpec((1, h), lambda i: (0, 0)),
        ],
        out_specs=pl.BlockSpec((blk, h), lambda i: (i, 0)),
        out_shape=jax.ShapeDtypeStruct((n, h), F32),
    )(x, agg_p, c0, c1, w1x, w1a, b1, g, b, w2, b2)


def kernel(x, edge_index, edge_attr, e_W1, e_b1, e_ln_g, e_ln_b, e_W2, e_b2,
           n_W1, n_b1, n_ln_g, n_ln_b, n_W2, n_b2):
    n, d = x.shape
    e, ed = edge_attr.shape
    h = e_W2.shape[1]

    per_w_chunks = -(-e // (NW * CHUNK))
    per_w_chunks += per_w_chunks % 2
    epad = NW * CHUNK * per_w_chunks
    pad = epad - e

    row = edge_index[0]
    col = edge_index[1]
    pad_g = jnp.arange(pad, dtype=jnp.int32) % jnp.int32(128)
    rowg = jnp.concatenate([row, pad_g])
    colg = jnp.concatenate([col, pad_g])
    pad_s = jnp.int32(n) + jnp.arange(pad, dtype=jnp.int32) % jnp.int32(16)
    cols = jnp.concatenate([col, pad_s])
    ea_pad = jnp.concatenate([edge_attr, jnp.zeros((pad, ed), F32)], axis=0)

    w1s = e_W1[:d]
    w1d = e_W1[d:2 * d]
    w1e = e_W1[2 * d:]

    npad = -(-(n + 16) // 128) * 128

    xs, xd = _tc_precompute(x, w1s, w1d)
    gs, gd = _sc_gather(xs, xd, rowg, colg, epad)
    cnt1d = _sc_scatter_cnt(cols, jnp.zeros((CHUNK,), F32),
                            jnp.ones((CHUNK,), F32), epad, npad)
    ean_pad = _tc_edge(gs, gd, ea_pad, w1e,
                       e_b1.reshape(1, -1), e_ln_g.reshape(1, -1),
                       e_ln_b.reshape(1, -1), e_W2, e_b2.reshape(1, -1))
    edge_attr_new = ean_pad[:e]

    agg_p = _sc_scatter_agg(ean_pad, cols, jnp.zeros((CHUNK, h), F32), npad)
    c0 = cnt1d[:npad].reshape(npad, 1)
    c1 = cnt1d[npad:].reshape(npad, 1)

    x_new = _tc_node(x, agg_p, c0, c1,
                     n_W1[:d], n_W1[d:],
                     n_b1.reshape(1, -1), n_ln_g.reshape(1, -1),
                     n_ln_b.reshape(1, -1), n_W2, n_b2.reshape(1, -1))
    return (x_new, edge_attr_new)


# speedup vs baseline: 1.0375x; 1.0005x over previous
import functools

import jax
import jax.numpy as jnp
from jax import lax
from jax.experimental import pallas as pl
from jax.experimental.pallas import tpu as pltpu
from jax.experimental.pallas import tpu_sc as plsc

F32 = jnp.float32
NC, NS = 2, 16
NW = NC * NS
CHUNK = 128
LN_EPS = 1e-5


def _sc_mesh():
    return plsc.VectorSubcoreMesh(core_axis_name="c", subcore_axis_name="s")


def _tc_precompute(x, w_s, w_d):
    n, d = x.shape
    blk = 2000
    h = w_s.shape[1]

    def body(x_ref, ws_ref, wd_ref, xs_ref, xd_ref):
        xb = x_ref[...]
        xs_ref[...] = jnp.dot(xb, ws_ref[...], preferred_element_type=F32)
        xd_ref[...] = jnp.dot(xb, wd_ref[...], preferred_element_type=F32)

    return pl.pallas_call(
        body,
        grid=(n // blk,),
        in_specs=[
            pl.BlockSpec((blk, d), lambda i: (i, 0)),
            pl.BlockSpec((d, h), lambda i: (0, 0)),
            pl.BlockSpec((d, h), lambda i: (0, 0)),
        ],
        out_specs=[
            pl.BlockSpec((blk, h), lambda i: (i, 0)),
            pl.BlockSpec((blk, h), lambda i: (i, 0)),
        ],
        out_shape=[
            jax.ShapeDtypeStruct((n, h), F32),
            jax.ShapeDtypeStruct((n, h), F32),
        ],
    )(x, w_s, w_d)


def _sc_gather(xs, xd, rowg, colg, epad):
    h = xs.shape[1]
    per_w = epad // NW
    nchunks = per_w // CHUNK

    @functools.partial(
        pl.kernel,
        out_type=(
            jax.ShapeDtypeStruct((epad, h), F32),
            jax.ShapeDtypeStruct((epad, h), F32),
        ),
        mesh=_sc_mesh(),
        scratch_types=[
            pltpu.VMEM((CHUNK,), jnp.int32),
            pltpu.VMEM((CHUNK,), jnp.int32),
            pltpu.VMEM((CHUNK,), jnp.int32),
            pltpu.VMEM((CHUNK,), jnp.int32),
            pltpu.VMEM((CHUNK, h), F32),
            pltpu.VMEM((CHUNK, h), F32),
            pltpu.VMEM((CHUNK, h), F32),
            pltpu.VMEM((CHUNK, h), F32),
        ] + [pltpu.SemaphoreType.DMA] * 12,
    )
    def k(xs_hbm, xd_hbm, row_hbm, col_hbm, gs_hbm, gd_hbm,
          ia0, ib0, ia1, ib1, ra0, rb0, ra1, rb1,
          s0, s1, s2, s3, s4, s5, s6, s7, s8, s9, s10, s11):
        wid = lax.axis_index("s") * NC + lax.axis_index("c")
        base_w = wid * per_w

        def drain_writes(b0, b1):
            pltpu.make_async_copy(ra0, gs_hbm.at[pl.ds(b0, CHUNK)], s8).wait()
            pltpu.make_async_copy(rb0, gd_hbm.at[pl.ds(b0, CHUNK)], s9).wait()
            pltpu.make_async_copy(ra1, gs_hbm.at[pl.ds(b1, CHUNK)], s10).wait()
            pltpu.make_async_copy(rb1, gd_hbm.at[pl.ds(b1, CHUNK)], s11).wait()

        def body(j, carry):
            b0 = base_w + (2 * j) * CHUNK
            b1 = b0 + CHUNK
            c1 = pltpu.async_copy(row_hbm.at[pl.ds(b0, CHUNK)], ia0, s0)
            c2 = pltpu.async_copy(col_hbm.at[pl.ds(b0, CHUNK)], ib0, s1)
            c3 = pltpu.async_copy(row_hbm.at[pl.ds(b1, CHUNK)], ia1, s2)
            c4 = pltpu.async_copy(col_hbm.at[pl.ds(b1, CHUNK)], ib1, s3)

            @pl.when(j > 0)
            def _():
                drain_writes(b0, b1)

            c1.wait()
            g1 = pltpu.async_copy(xs_hbm.at[ia0], ra0, s4)
            c2.wait()
            g2 = pltpu.async_copy(xd_hbm.at[ib0], rb0, s5)
            c3.wait()
            g3 = pltpu.async_copy(xs_hbm.at[ia1], ra1, s6)
            c4.wait()
            g4 = pltpu.async_copy(xd_hbm.at[ib1], rb1, s7)
            g1.wait()
            pltpu.async_copy(ra0, gs_hbm.at[pl.ds(b0, CHUNK)], s8)
            g2.wait()
            pltpu.async_copy(rb0, gd_hbm.at[pl.ds(b0, CHUNK)], s9)
            g3.wait()
            pltpu.async_copy(ra1, gs_hbm.at[pl.ds(b1, CHUNK)], s10)
            g4.wait()
            pltpu.async_copy(rb1, gd_hbm.at[pl.ds(b1, CHUNK)], s11)
            return carry

        lax.fori_loop(0, nchunks // 2, body, 0)
        drain_writes(base_w, base_w + CHUNK)

    return k(xs, xd, rowg, colg)


def _tc_edge(gs, gd, ea, w1e, b1, g, b, w2, b2):
    epad, h = gs.shape
    ed = ea.shape[1]
    blk = 2048

    def body(gs_ref, gd_ref, ea_ref, w1e_ref, b1_ref, g_ref, b_ref,
             w2_ref, b2_ref, out_ref):
        pre = (gs_ref[...] + gd_ref[...]
               + jnp.dot(ea_ref[...], w1e_ref[...], preferred_element_type=F32)
               + b1_ref[...])
        hh = jnp.maximum(pre, 0.0)
        m = jnp.mean(hh, axis=-1, keepdims=True)
        c = hh - m
        v = jnp.mean(c * c, axis=-1, keepdims=True)
        hn = c * lax.rsqrt(v + LN_EPS) * g_ref[...] + b_ref[...]
        out_ref[...] = (jnp.dot(hn, w2_ref[...], preferred_element_type=F32)
                        + b2_ref[...])

    return pl.pallas_call(
        body,
        grid=(epad // blk,),
        in_specs=[
            pl.BlockSpec((blk, h), lambda i: (i, 0)),
            pl.BlockSpec((blk, h), lambda i: (i, 0)),
            pl.BlockSpec((blk, ed), lambda i: (i, 0)),
            pl.BlockSpec((ed, h), lambda i: (0, 0)),
            pl.BlockSpec((1, h), lambda i: (0, 0)),
            pl.BlockSpec((1, h), lambda i: (0, 0)),
            pl.BlockSpec((1, h), lambda i: (0, 0)),
            pl.BlockSpec((h, h), lambda i: (0, 0)),
            pl.BlockSpec((1, h), lambda i: (0, 0)),
        ],
        out_specs=pl.BlockSpec((blk, h), lambda i: (i, 0)),
        out_shape=jax.ShapeDtypeStruct((epad, h), F32),
    )(gs, gd, ea, w1e, b1, g, b, w2, b2)


def _row_spans(rows_per_tile):
    spans = []
    off = 0
    while off < rows_per_tile:
        c = min(CHUNK, rows_per_tile - off)
        spans.append((off, c))
        off += c
    return spans


def _sc_scatter_agg(ean, cols, zeros_agg, npad):
    epad, h = ean.shape
    per_w = epad // NW
    nchunks = per_w // CHUNK
    rows_per_tile = npad // NS
    spans = _row_spans(rows_per_tile)

    @functools.partial(
        pl.kernel,
        out_type=jax.ShapeDtypeStruct((NC, npad, h), F32),
        mesh=_sc_mesh(),
        scratch_types=[
            pltpu.VMEM((CHUNK,), jnp.int32),
            pltpu.VMEM((CHUNK,), jnp.int32),
            pltpu.VMEM((CHUNK, h), F32),
            pltpu.VMEM((CHUNK, h), F32),
            pltpu.VMEM_SHARED((npad, h), F32),
        ] + [pltpu.SemaphoreType.DMA] * 6,
    )
    def k(ean_hbm, col_hbm, za_hbm, agg_hbm, idx0, idx1, rows0, rows1,
          sh_agg, s0, s1, s2, s3, s4, s5):
        cid = lax.axis_index("c")
        sid = lax.axis_index("s")
        wid = sid * NC + cid

        pltpu.sync_copy(za_hbm, rows0)
        for joff, jlen in spans:
            zbase = sid * rows_per_tile + joff
            pltpu.sync_copy(rows0.at[pl.ds(0, jlen)],
                            sh_agg.at[pl.ds(zbase, jlen)])
        plsc.subcore_barrier()

        base_w = wid * per_w

        def drain_adds():
            pltpu.make_async_copy(rows0, sh_agg.at[idx0], s4).wait()
            pltpu.make_async_copy(rows1, sh_agg.at[idx1], s5).wait()

        def body(j, carry):
            b0 = base_w + (2 * j) * CHUNK
            b1 = b0 + CHUNK
            @pl.when(j > 0)
            def _():
                drain_adds()

            c0 = pltpu.async_copy(col_hbm.at[pl.ds(b0, CHUNK)], idx0, s0)
            c1 = pltpu.async_copy(col_hbm.at[pl.ds(b1, CHUNK)], idx1, s1)
            r0 = pltpu.async_copy(ean_hbm.at[pl.ds(b0, CHUNK)], rows0, s2)
            r1 = pltpu.async_copy(ean_hbm.at[pl.ds(b1, CHUNK)], rows1, s3)
            c0.wait()
            r0.wait()
            pltpu.async_copy(rows0, sh_agg.at[idx0], s4, add=True)
            c1.wait()
            r1.wait()
            pltpu.async_copy(rows1, sh_agg.at[idx1], s5, add=True)
            return carry

        lax.fori_loop(0, nchunks // 2, body, 0)
        drain_adds()
        plsc.subcore_barrier()

        def copy_out(c):
            for joff, jlen in spans:
                obase = sid * rows_per_tile + joff
                pltpu.sync_copy(sh_agg.at[pl.ds(obase, jlen)],
                                rows0.at[pl.ds(0, jlen)])
                pltpu.sync_copy(rows0.at[pl.ds(0, jlen)],
                                agg_hbm.at[c, pl.ds(obase, jlen)])

        @pl.when(cid == 0)
        def _():
            copy_out(0)

        @pl.when(cid == 1)
        def _():
            copy_out(1)

    return k(ean, cols, zeros_agg)


def _sc_scatter_cnt(cols, zeros1, ones1, epad, npad):
    per_w = epad // NW
    nchunks = per_w // CHUNK
    rows_per_tile = npad // NS
    spans = _row_spans(rows_per_tile)

    @functools.partial(
        pl.kernel,
        out_type=jax.ShapeDtypeStruct((NC * npad,), F32),
        mesh=_sc_mesh(),
        scratch_types=[
            pltpu.VMEM((CHUNK,), jnp.int32),
            pltpu.VMEM((CHUNK,), F32),
            pltpu.VMEM((CHUNK,), F32),
            pltpu.VMEM_SHARED((npad,), F32),
        ],
    )
    def k(col_hbm, z_hbm, ones_hbm, cnt_hbm, idx_v, stage_v, ones_v, sh_cnt):
        cid = lax.axis_index("c")
        sid = lax.axis_index("s")
        wid = sid * NC + cid

        pltpu.sync_copy(z_hbm, stage_v)
        for joff, jlen in spans:
            zbase = sid * rows_per_tile + joff
            pltpu.sync_copy(stage_v.at[pl.ds(0, jlen)],
                            sh_cnt.at[pl.ds(zbase, jlen)])
        pltpu.sync_copy(ones_hbm, ones_v)
        plsc.subcore_barrier()

        base_w = wid * per_w

        def body(i, carry):
            base = base_w + i * CHUNK
            pltpu.sync_copy(col_hbm.at[pl.ds(base, CHUNK)], idx_v)
            pltpu.sync_copy(ones_v, sh_cnt.at[idx_v], add=True)
            return carry

        lax.fori_loop(0, nchunks, body, 0)
        plsc.subcore_barrier()

        def copy_out(c):
            for joff, jlen in spans:
                obase = sid * rows_per_tile + joff
                pltpu.sync_copy(sh_cnt.at[pl.ds(obase, jlen)],
                                stage_v.at[pl.ds(0, jlen)])
                pltpu.sync_copy(stage_v.at[pl.ds(0, jlen)],
                                cnt_hbm.at[pl.ds(c * npad + obase, jlen)])

        @pl.when(cid == 0)
        def _():
            copy_out(0)

        @pl.when(cid == 1)
        def _():
            copy_out(1)

    return k(cols, zeros1, ones1)


def _tc_node(x, agg_p, c0, c1, w1x, w1a, b1, g, b, w2, b2):
    n, d = x.shape
    h = w1x.shape[1]
    cw = c0.shape[1]
    blk = 2000

    def body(x_ref, a_ref, c0_ref, c1_ref, w1x_ref, w1a_ref,
             b1_ref, g_ref, b_ref, w2_ref, b2_ref, out_ref):
        cnt = c0_ref[...][:, :1] + c1_ref[...][:, :1]
        inv = 1.0 / jnp.maximum(cnt, 1.0)
        agg = (a_ref[0] + a_ref[1]) * inv
        gg = (jnp.dot(x_ref[...], w1x_ref[...], preferred_element_type=F32)
              + jnp.dot(agg, w1a_ref[...], preferred_element_type=F32)
              + b1_ref[...])
        hh_ = jnp.maximum(gg, 0.0)
        m = jnp.mean(hh_, axis=-1, keepdims=True)
        c = hh_ - m
        v = jnp.mean(c * c, axis=-1, keepdims=True)
        hn = c * lax.rsqrt(v + LN_EPS) * g_ref[...] + b_ref[...]
        out_ref[...] = (jnp.dot(hn, w2_ref[...], preferred_element_type=F32)
                        + b2_ref[...])

    return pl.pallas_call(
        body,
        grid=(n // blk,),
        in_specs=[
            pl.BlockSpec((blk, d), lambda i: (i, 0)),
            pl.BlockSpec((NC, blk, h), lambda i: (0, i, 0)),
            pl.BlockSpec((blk, cw), lambda i: (i, 0)),
            pl.BlockSpec((blk, cw), lambda i: (i, 0)),
            pl.BlockSpec((d, h), lambda i: (0, 0)),
            pl.BlockSpec((h, h), lambda i: (0, 0)),
            pl.BlockSpec((1, h), lambda i: (0, 0)),
            pl.BlockSpec((1, h), lambda i: (0, 0)),
            pl.BlockSpec((1, h), lambda i: (0, 0)),
            pl.BlockSpec((h, h), lambda i: (0, 0)),
            pl.BlockSpec((1, h), lambda i: (0, 0)),
        ],
        out_specs=pl.BlockSpec((blk, h), lambda i: (i, 0)),
        out_shape=jax.ShapeDtypeStruct((n, h), F32),
    )(x, agg_p, c0, c1, w1x, w1a, b1, g, b, w2, b2)


def kernel(x, edge_index, edge_attr, e_W1, e_b1, e_ln_g, e_ln_b, e_W2, e_b2,
           n_W1, n_b1, n_ln_g, n_ln_b, n_W2, n_b2):
    n, d = x.shape
    e, ed = edge_attr.shape
    h = e_W2.shape[1]

    per_w_chunks = -(-e // (NW * CHUNK))
    per_w_chunks += per_w_chunks % 2
    epad = NW * CHUNK * per_w_chunks
    pad = epad - e

    row = edge_index[0]
    col = edge_index[1]
    pad_g = jnp.arange(pad, dtype=jnp.int32) % jnp.int32(128)
    rowg = jnp.concatenate([row, pad_g])
    colg = jnp.concatenate([col, pad_g])
    pad_s = jnp.int32(n) + jnp.arange(pad, dtype=jnp.int32) % jnp.int32(16)
    cols = jnp.concatenate([col, pad_s])
    ea_pad = jnp.concatenate([edge_attr, jnp.zeros((pad, ed), F32)], axis=0)

    w1s = e_W1[:d]
    w1d = e_W1[d:2 * d]
    w1e = e_W1[2 * d:]

    npad = -(-(n + 16) // 128) * 128

    xs, xd = _tc_precompute(x, w1s, w1d)
    gs, gd = _sc_gather(xs, xd, rowg, colg, epad)
    cnt1d = _sc_scatter_cnt(cols, jnp.zeros((CHUNK,), F32),
                            jnp.ones((CHUNK,), F32), epad, npad)
    ean_pad = _tc_edge(gs, gd, ea_pad, w1e,
                       e_b1.reshape(1, -1), e_ln_g.reshape(1, -1),
                       e_ln_b.reshape(1, -1), e_W2, e_b2.reshape(1, -1))
    edge_attr_new = ean_pad[:e]

    agg_p = _sc_scatter_agg(ean_pad, cols, jnp.zeros((CHUNK, h), F32), npad)
    c0 = cnt1d[:npad].reshape(npad, 1)
    c1 = cnt1d[npad:].reshape(npad, 1)

    x_new = _tc_node(x, agg_p, c0, c1,
                     n_W1[:d], n_W1[d:],
                     n_b1.reshape(1, -1), n_ln_g.reshape(1, -1),
                     n_ln_b.reshape(1, -1), n_W2, n_b2.reshape(1, -1))
    return (x_new, edge_attr_new)


# speedup vs baseline: 1.0420x; 1.0043x over previous
import functools

import jax
import jax.numpy as jnp
from jax import lax
from jax.experimental import pallas as pl
from jax.experimental.pallas import tpu as pltpu
from jax.experimental.pallas import tpu_sc as plsc

F32 = jnp.float32
NC, NS = 2, 16
NW = NC * NS
CHUNK = 128
LN_EPS = 1e-5


def _sc_mesh():
    return plsc.VectorSubcoreMesh(core_axis_name="c", subcore_axis_name="s")


def _tc_precompute(x, w_s, w_d):
    n, d = x.shape
    blk = 2000
    h = w_s.shape[1]

    def body(x_ref, ws_ref, wd_ref, xs_ref, xd_ref):
        xb = x_ref[...]
        xs_ref[...] = jnp.dot(xb, ws_ref[...], preferred_element_type=F32)
        xd_ref[...] = jnp.dot(xb, wd_ref[...], preferred_element_type=F32)

    return pl.pallas_call(
        body,
        grid=(n // blk,),
        in_specs=[
            pl.BlockSpec((blk, d), lambda i: (i, 0)),
            pl.BlockSpec((d, h), lambda i: (0, 0)),
            pl.BlockSpec((d, h), lambda i: (0, 0)),
        ],
        out_specs=[
            pl.BlockSpec((blk, h), lambda i: (i, 0)),
            pl.BlockSpec((blk, h), lambda i: (i, 0)),
        ],
        out_shape=[
            jax.ShapeDtypeStruct((n, h), F32),
            jax.ShapeDtypeStruct((n, h), F32),
        ],
    )(x, w_s, w_d)


def _sc_gather(xs, xd, rowg, colg, epad):
    h = xs.shape[1]
    per_w = epad // NW
    gchunk = 2 * CHUNK
    nchunks = per_w // gchunk

    @functools.partial(
        pl.kernel,
        out_type=(
            jax.ShapeDtypeStruct((epad, h), F32),
            jax.ShapeDtypeStruct((epad, h), F32),
        ),
        mesh=_sc_mesh(),
        scratch_types=[
            pltpu.VMEM((gchunk,), jnp.int32),
            pltpu.VMEM((gchunk,), jnp.int32),
            pltpu.VMEM((gchunk, h), F32),
            pltpu.VMEM((gchunk, h), F32),
        ] + [pltpu.SemaphoreType.DMA] * 6,
    )
    def k(xs_hbm, xd_hbm, row_hbm, col_hbm, gs_hbm, gd_hbm,
          ia, ib, ra, rb, s0, s1, s4, s5, s8, s9):
        wid = lax.axis_index("s") * NC + lax.axis_index("c")
        base_w = wid * per_w

        def drain_writes(b0):
            pltpu.make_async_copy(ra, gs_hbm.at[pl.ds(b0, gchunk)], s8).wait()
            pltpu.make_async_copy(rb, gd_hbm.at[pl.ds(b0, gchunk)], s9).wait()

        def body(j, carry):
            b0 = base_w + j * gchunk
            c1 = pltpu.async_copy(row_hbm.at[pl.ds(b0, gchunk)], ia, s0)
            c2 = pltpu.async_copy(col_hbm.at[pl.ds(b0, gchunk)], ib, s1)

            @pl.when(j > 0)
            def _():
                drain_writes(b0)

            c1.wait()
            g1 = pltpu.async_copy(xs_hbm.at[ia], ra, s4)
            c2.wait()
            g2 = pltpu.async_copy(xd_hbm.at[ib], rb, s5)
            g1.wait()
            pltpu.async_copy(ra, gs_hbm.at[pl.ds(b0, gchunk)], s8)
            g2.wait()
            pltpu.async_copy(rb, gd_hbm.at[pl.ds(b0, gchunk)], s9)
            return carry

        lax.fori_loop(0, nchunks, body, 0)
        drain_writes(base_w)

    return k(xs, xd, rowg, colg)


def _tc_edge(gs, gd, ea, w1e, b1, g, b, w2, b2):
    epad, h = gs.shape
    ed = ea.shape[1]
    blk = 2048

    def body(gs_ref, gd_ref, ea_ref, w1e_ref, b1_ref, g_ref, b_ref,
             w2_ref, b2_ref, out_ref):
        pre = (gs_ref[...] + gd_ref[...]
               + jnp.dot(ea_ref[...], w1e_ref[...], preferred_element_type=F32)
               + b1_ref[...])
        hh = jnp.maximum(pre, 0.0)
        m = jnp.mean(hh, axis=-1, keepdims=True)
        c = hh - m
        v = jnp.mean(c * c, axis=-1, keepdims=True)
        hn = c * lax.rsqrt(v + LN_EPS) * g_ref[...] + b_ref[...]
        out_ref[...] = (jnp.dot(hn, w2_ref[...], preferred_element_type=F32)
                        + b2_ref[...])

    return pl.pallas_call(
        body,
        grid=(epad // blk,),
        in_specs=[
            pl.BlockSpec((blk, h), lambda i: (i, 0)),
            pl.BlockSpec((blk, h), lambda i: (i, 0)),
            pl.BlockSpec((blk, ed), lambda i: (i, 0)),
            pl.BlockSpec((ed, h), lambda i: (0, 0)),
            pl.BlockSpec((1, h), lambda i: (0, 0)),
            pl.BlockSpec((1, h), lambda i: (0, 0)),
            pl.BlockSpec((1, h), lambda i: (0, 0)),
            pl.BlockSpec((h, h), lambda i: (0, 0)),
            pl.BlockSpec((1, h), lambda i: (0, 0)),
        ],
        out_specs=pl.BlockSpec((blk, h), lambda i: (i, 0)),
        out_shape=jax.ShapeDtypeStruct((epad, h), F32),
    )(gs, gd, ea, w1e, b1, g, b, w2, b2)


def _row_spans(rows_per_tile):
    spans = []
    off = 0
    while off < rows_per_tile:
        c = min(CHUNK, rows_per_tile - off)
        spans.append((off, c))
        off += c
    return spans


def _sc_scatter_agg(ean, cols, zeros_agg, npad):
    epad, h = ean.shape
    per_w = epad // NW
    nchunks = per_w // CHUNK
    rows_per_tile = npad // NS
    spans = _row_spans(rows_per_tile)

    @functools.partial(
        pl.kernel,
        out_type=jax.ShapeDtypeStruct((NC, npad, h), F32),
        mesh=_sc_mesh(),
        scratch_types=[
            pltpu.VMEM((CHUNK,), jnp.int32),
            pltpu.VMEM((CHUNK,), jnp.int32),
            pltpu.VMEM((CHUNK, h), F32),
            pltpu.VMEM((CHUNK, h), F32),
            pltpu.VMEM_SHARED((npad, h), F32),
        ] + [pltpu.SemaphoreType.DMA] * 6,
    )
    def k(ean_hbm, col_hbm, za_hbm, agg_hbm, idx0, idx1, rows0, rows1,
          sh_agg, s0, s1, s2, s3, s4, s5):
        cid = lax.axis_index("c")
        sid = lax.axis_index("s")
        wid = sid * NC + cid

        pltpu.sync_copy(za_hbm, rows0)
        for joff, jlen in spans:
            zbase = sid * rows_per_tile + joff
            pltpu.sync_copy(rows0.at[pl.ds(0, jlen)],
                            sh_agg.at[pl.ds(zbase, jlen)])
        plsc.subcore_barrier()

        base_w = wid * per_w

        def drain_adds():
            pltpu.make_async_copy(rows0, sh_agg.at[idx0], s4).wait()
            pltpu.make_async_copy(rows1, sh_agg.at[idx1], s5).wait()

        def body(j, carry):
            b0 = base_w + (2 * j) * CHUNK
            b1 = b0 + CHUNK
            @pl.when(j > 0)
            def _():
                drain_adds()

            c0 = pltpu.async_copy(col_hbm.at[pl.ds(b0, CHUNK)], idx0, s0)
            c1 = pltpu.async_copy(col_hbm.at[pl.ds(b1, CHUNK)], idx1, s1)
            r0 = pltpu.async_copy(ean_hbm.at[pl.ds(b0, CHUNK)], rows0, s2)
            r1 = pltpu.async_copy(ean_hbm.at[pl.ds(b1, CHUNK)], rows1, s3)
            c0.wait()
            r0.wait()
            pltpu.async_copy(rows0, sh_agg.at[idx0], s4, add=True)
            c1.wait()
            r1.wait()
            pltpu.async_copy(rows1, sh_agg.at[idx1], s5, add=True)
            return carry

        lax.fori_loop(0, nchunks // 2, body, 0)
        drain_adds()
        plsc.subcore_barrier()

        def copy_out(c):
            for joff, jlen in spans:
                obase = sid * rows_per_tile + joff
                pltpu.sync_copy(sh_agg.at[pl.ds(obase, jlen)],
                                rows0.at[pl.ds(0, jlen)])
                pltpu.sync_copy(rows0.at[pl.ds(0, jlen)],
                                agg_hbm.at[c, pl.ds(obase, jlen)])

        @pl.when(cid == 0)
        def _():
            copy_out(0)

        @pl.when(cid == 1)
        def _():
            copy_out(1)

    return k(ean, cols, zeros_agg)


def _sc_scatter_cnt(cols, zeros1, ones1, epad, npad):
    per_w = epad // NW
    nchunks = per_w // CHUNK
    rows_per_tile = npad // NS
    spans = _row_spans(rows_per_tile)

    @functools.partial(
        pl.kernel,
        out_type=jax.ShapeDtypeStruct((NC * npad,), F32),
        mesh=_sc_mesh(),
        scratch_types=[
            pltpu.VMEM((CHUNK,), jnp.int32),
            pltpu.VMEM((CHUNK,), F32),
            pltpu.VMEM((CHUNK,), F32),
            pltpu.VMEM_SHARED((npad,), F32),
        ],
    )
    def k(col_hbm, z_hbm, ones_hbm, cnt_hbm, idx_v, stage_v, ones_v, sh_cnt):
        cid = lax.axis_index("c")
        sid = lax.axis_index("s")
        wid = sid * NC + cid

        pltpu.sync_copy(z_hbm, stage_v)
        for joff, jlen in spans:
            zbase = sid * rows_per_tile + joff
            pltpu.sync_copy(stage_v.at[pl.ds(0, jlen)],
                            sh_cnt.at[pl.ds(zbase, jlen)])
        pltpu.sync_copy(ones_hbm, ones_v)
        plsc.subcore_barrier()

        base_w = wid * per_w

        def body(i, carry):
            base = base_w + i * CHUNK
            pltpu.sync_copy(col_hbm.at[pl.ds(base, CHUNK)], idx_v)
            pltpu.sync_copy(ones_v, sh_cnt.at[idx_v], add=True)
            return carry

        lax.fori_loop(0, nchunks, body, 0)
        plsc.subcore_barrier()

        def copy_out(c):
            for joff, jlen in spans:
                obase = sid * rows_per_tile + joff
                pltpu.sync_copy(sh_cnt.at[pl.ds(obase, jlen)],
                                stage_v.at[pl.ds(0, jlen)])
                pltpu.sync_copy(stage_v.at[pl.ds(0, jlen)],
                                cnt_hbm.at[pl.ds(c * npad + obase, jlen)])

        @pl.when(cid == 0)
        def _():
            copy_out(0)

        @pl.when(cid == 1)
        def _():
            copy_out(1)

    return k(cols, zeros1, ones1)


def _tc_node(x, agg_p, c0, c1, w1x, w1a, b1, g, b, w2, b2):
    n, d = x.shape
    h = w1x.shape[1]
    cw = c0.shape[1]
    blk = 2000

    def body(x_ref, a_ref, c0_ref, c1_ref, w1x_ref, w1a_ref,
             b1_ref, g_ref, b_ref, w2_ref, b2_ref, out_ref):
        cnt = c0_ref[...][:, :1] + c1_ref[...][:, :1]
        inv = 1.0 / jnp.maximum(cnt, 1.0)
        agg = (a_ref[0] + a_ref[1]) * inv
        gg = (jnp.dot(x_ref[...], w1x_ref[...], preferred_element_type=F32)
              + jnp.dot(agg, w1a_ref[...], preferred_element_type=F32)
              + b1_ref[...])
        hh_ = jnp.maximum(gg, 0.0)
        m = jnp.mean(hh_, axis=-1, keepdims=True)
        c = hh_ - m
        v = jnp.mean(c * c, axis=-1, keepdims=True)
        hn = c * lax.rsqrt(v + LN_EPS) * g_ref[...] + b_ref[...]
        out_ref[...] = (jnp.dot(hn, w2_ref[...], preferred_element_type=F32)
                        + b2_ref[...])

    return pl.pallas_call(
        body,
        grid=(n // blk,),
        in_specs=[
            pl.BlockSpec((blk, d), lambda i: (i, 0)),
            pl.BlockSpec((NC, blk, h), lambda i: (0, i, 0)),
            pl.BlockSpec((blk, cw), lambda i: (i, 0)),
            pl.BlockSpec((blk, cw), lambda i: (i, 0)),
            pl.BlockSpec((d, h), lambda i: (0, 0)),
            pl.BlockSpec((h, h), lambda i: (0, 0)),
            pl.BlockSpec((1, h), lambda i: (0, 0)),
            pl.BlockSpec((1, h), lambda i: (0, 0)),
            pl.BlockSpec((1, h), lambda i: (0, 0)),
            pl.BlockSpec((h, h), lambda i: (0, 0)),
            pl.BlockSpec((1, h), lambda i: (0, 0)),
        ],
        out_specs=pl.BlockSpec((blk, h), lambda i: (i, 0)),
        out_shape=jax.ShapeDtypeStruct((n, h), F32),
    )(x, agg_p, c0, c1, w1x, w1a, b1, g, b, w2, b2)


def kernel(x, edge_index, edge_attr, e_W1, e_b1, e_ln_g, e_ln_b, e_W2, e_b2,
           n_W1, n_b1, n_ln_g, n_ln_b, n_W2, n_b2):
    n, d = x.shape
    e, ed = edge_attr.shape
    h = e_W2.shape[1]

    per_w_chunks = -(-e // (NW * CHUNK))
    per_w_chunks += per_w_chunks % 2
    epad = NW * CHUNK * per_w_chunks
    pad = epad - e

    row = edge_index[0]
    col = edge_index[1]
    pad_g = jnp.arange(pad, dtype=jnp.int32) % jnp.int32(128)
    rowg = jnp.concatenate([row, pad_g])
    colg = jnp.concatenate([col, pad_g])
    pad_s = jnp.int32(n) + jnp.arange(pad, dtype=jnp.int32) % jnp.int32(16)
    cols = jnp.concatenate([col, pad_s])
    ea_pad = jnp.concatenate([edge_attr, jnp.zeros((pad, ed), F32)], axis=0)

    w1s = e_W1[:d]
    w1d = e_W1[d:2 * d]
    w1e = e_W1[2 * d:]

    npad = -(-(n + 16) // 128) * 128

    xs, xd = _tc_precompute(x, w1s, w1d)
    gs, gd = _sc_gather(xs, xd, rowg, colg, epad)
    cnt1d = _sc_scatter_cnt(cols, jnp.zeros((CHUNK,), F32),
                            jnp.ones((CHUNK,), F32), epad, npad)
    ean_pad = _tc_edge(gs, gd, ea_pad, w1e,
                       e_b1.reshape(1, -1), e_ln_g.reshape(1, -1),
                       e_ln_b.reshape(1, -1), e_W2, e_b2.reshape(1, -1))
    edge_attr_new = ean_pad[:e]

    agg_p = _sc_scatter_agg(ean_pad, cols, jnp.zeros((CHUNK, h), F32), npad)
    c0 = cnt1d[:npad].reshape(npad, 1)
    c1 = cnt1d[npad:].reshape(npad, 1)

    x_new = _tc_node(x, agg_p, c0, c1,
                     n_W1[:d], n_W1[d:],
                     n_b1.reshape(1, -1), n_ln_g.reshape(1, -1),
                     n_ln_b.reshape(1, -1), n_W2, n_b2.reshape(1, -1))
    return (x_new, edge_attr_new)


# speedup vs baseline: 1.0426x; 1.0006x over previous
import functools

import jax
import jax.numpy as jnp
from jax import lax
from jax.experimental import pallas as pl
from jax.experimental.pallas import tpu as pltpu
from jax.experimental.pallas import tpu_sc as plsc

F32 = jnp.float32
NC, NS = 2, 16
NW = NC * NS
CHUNK = 128
LN_EPS = 1e-5


def _sc_mesh():
    return plsc.VectorSubcoreMesh(core_axis_name="c", subcore_axis_name="s")


def _tc_precompute(x, w_s, w_d):
    n, d = x.shape
    blk = 2000
    h = w_s.shape[1]

    def body(x_ref, ws_ref, wd_ref, xs_ref, xd_ref):
        xb = x_ref[...]
        xs_ref[...] = jnp.dot(xb, ws_ref[...], preferred_element_type=F32)
        xd_ref[...] = jnp.dot(xb, wd_ref[...], preferred_element_type=F32)

    return pl.pallas_call(
        body,
        grid=(n // blk,),
        in_specs=[
            pl.BlockSpec((blk, d), lambda i: (i, 0)),
            pl.BlockSpec((d, h), lambda i: (0, 0)),
            pl.BlockSpec((d, h), lambda i: (0, 0)),
        ],
        out_specs=[
            pl.BlockSpec((blk, h), lambda i: (i, 0)),
            pl.BlockSpec((blk, h), lambda i: (i, 0)),
        ],
        out_shape=[
            jax.ShapeDtypeStruct((n, h), F32),
            jax.ShapeDtypeStruct((n, h), F32),
        ],
    )(x, w_s, w_d)


def _sc_gather(xs, xd, rowg, colg, epad):
    h = xs.shape[1]
    per_w = epad // NW
    gchunk = 2 * CHUNK
    nchunks = per_w // gchunk

    @functools.partial(
        pl.kernel,
        out_type=(
            jax.ShapeDtypeStruct((epad, h), F32),
            jax.ShapeDtypeStruct((epad, h), F32),
        ),
        mesh=_sc_mesh(),
        scratch_types=[
            pltpu.VMEM((gchunk,), jnp.int32),
            pltpu.VMEM((gchunk,), jnp.int32),
            pltpu.VMEM((gchunk, h), F32),
            pltpu.VMEM((gchunk, h), F32),
        ] + [pltpu.SemaphoreType.DMA] * 6,
    )
    def k(xs_hbm, xd_hbm, row_hbm, col_hbm, gs_hbm, gd_hbm,
          ia, ib, ra, rb, s0, s1, s4, s5, s8, s9):
        wid = lax.axis_index("s") * NC + lax.axis_index("c")
        base_w = wid * per_w

        def drain_writes(b0):
            pltpu.make_async_copy(ra, gs_hbm.at[pl.ds(b0, gchunk)], s8).wait()
            pltpu.make_async_copy(rb, gd_hbm.at[pl.ds(b0, gchunk)], s9).wait()

        def body(j, carry):
            b0 = base_w + j * gchunk
            c1 = pltpu.async_copy(row_hbm.at[pl.ds(b0, gchunk)], ia, s0)
            c2 = pltpu.async_copy(col_hbm.at[pl.ds(b0, gchunk)], ib, s1)

            @pl.when(j > 0)
            def _():
                drain_writes(b0)

            c1.wait()
            g1 = pltpu.async_copy(xs_hbm.at[ia], ra, s4)
            c2.wait()
            g2 = pltpu.async_copy(xd_hbm.at[ib], rb, s5)
            g1.wait()
            pltpu.async_copy(ra, gs_hbm.at[pl.ds(b0, gchunk)], s8)
            g2.wait()
            pltpu.async_copy(rb, gd_hbm.at[pl.ds(b0, gchunk)], s9)
            return carry

        lax.fori_loop(0, nchunks, body, 0)
        drain_writes(base_w)

    return k(xs, xd, rowg, colg)


def _tc_edge(gs, gd, ea, w1e, b1, g, b, w2, b2):
    epad, h = gs.shape
    ed = ea.shape[1]
    blk = 2048

    def body(gs_ref, gd_ref, ea_ref, w1e_ref, b1_ref, g_ref, b_ref,
             w2_ref, b2_ref, out_ref):
        pre = (gs_ref[...] + gd_ref[...]
               + jnp.dot(ea_ref[...], w1e_ref[...], preferred_element_type=F32)
               + b1_ref[...])
        hh = jnp.maximum(pre, 0.0)
        m = jnp.mean(hh, axis=-1, keepdims=True)
        c = hh - m
        v = jnp.mean(c * c, axis=-1, keepdims=True)
        hn = c * lax.rsqrt(v + LN_EPS) * g_ref[...] + b_ref[...]
        out_ref[...] = (jnp.dot(hn, w2_ref[...], preferred_element_type=F32)
                        + b2_ref[...])

    return pl.pallas_call(
        body,
        grid=(epad // blk,),
        in_specs=[
            pl.BlockSpec((blk, h), lambda i: (i, 0)),
            pl.BlockSpec((blk, h), lambda i: (i, 0)),
            pl.BlockSpec((blk, ed), lambda i: (i, 0)),
            pl.BlockSpec((ed, h), lambda i: (0, 0)),
            pl.BlockSpec((1, h), lambda i: (0, 0)),
            pl.BlockSpec((1, h), lambda i: (0, 0)),
            pl.BlockSpec((1, h), lambda i: (0, 0)),
            pl.BlockSpec((h, h), lambda i: (0, 0)),
            pl.BlockSpec((1, h), lambda i: (0, 0)),
        ],
        out_specs=pl.BlockSpec((blk, h), lambda i: (i, 0)),
        out_shape=jax.ShapeDtypeStruct((epad, h), F32),
    )(gs, gd, ea, w1e, b1, g, b, w2, b2)


def _row_spans(rows_per_tile):
    spans = []
    off = 0
    while off < rows_per_tile:
        c = min(CHUNK, rows_per_tile - off)
        spans.append((off, c))
        off += c
    return spans


def _sc_scatter_agg(ean, cols, zeros_agg, npad):
    epad, h = ean.shape
    per_w = epad // NW
    schunk = CHUNK
    nchunks = per_w // schunk
    rows_per_tile = npad // NS
    spans = _row_spans(rows_per_tile)

    @functools.partial(
        pl.kernel,
        out_type=jax.ShapeDtypeStruct((NC, npad, h), F32),
        mesh=_sc_mesh(),
        scratch_types=[
            pltpu.VMEM((schunk,), jnp.int32),
            pltpu.VMEM((schunk,), jnp.int32),
            pltpu.VMEM((schunk, h), F32),
            pltpu.VMEM((schunk, h), F32),
            pltpu.VMEM_SHARED((npad, h), F32),
        ] + [pltpu.SemaphoreType.DMA] * 6,
    )
    def k(ean_hbm, col_hbm, za_hbm, agg_hbm, idx0, idx1, rows0, rows1,
          sh_agg, s0, s1, s2, s3, s4, s5):
        cid = lax.axis_index("c")
        sid = lax.axis_index("s")
        wid = sid * NC + cid

        pltpu.sync_copy(za_hbm, rows0)
        for joff, jlen in spans:
            zbase = sid * rows_per_tile + joff
            pltpu.sync_copy(rows0.at[pl.ds(0, jlen)],
                            sh_agg.at[pl.ds(zbase, jlen)])
        plsc.subcore_barrier()

        base_w = wid * per_w

        def drain_adds():
            pltpu.make_async_copy(rows0, sh_agg.at[idx0], s4).wait()
            pltpu.make_async_copy(rows1, sh_agg.at[idx1], s5).wait()

        def body(j, carry):
            b0 = base_w + (2 * j) * schunk
            b1 = b0 + schunk
            @pl.when(j > 0)
            def _():
                drain_adds()

            c0 = pltpu.async_copy(col_hbm.at[pl.ds(b0, schunk)], idx0, s0)
            c1 = pltpu.async_copy(col_hbm.at[pl.ds(b1, schunk)], idx1, s1)
            r0 = pltpu.async_copy(ean_hbm.at[pl.ds(b0, schunk)], rows0, s2)
            r1 = pltpu.async_copy(ean_hbm.at[pl.ds(b1, schunk)], rows1, s3)
            c0.wait()
            r0.wait()
            pltpu.async_copy(rows0, sh_agg.at[idx0], s4, add=True)
            c1.wait()
            r1.wait()
            pltpu.async_copy(rows1, sh_agg.at[idx1], s5, add=True)
            return carry

        lax.fori_loop(0, nchunks // 2, body, 0)
        drain_adds()
        plsc.subcore_barrier()

        def copy_out(c):
            for joff, jlen in spans:
                obase = sid * rows_per_tile + joff
                pltpu.sync_copy(sh_agg.at[pl.ds(obase, jlen)],
                                rows0.at[pl.ds(0, jlen)])
                pltpu.sync_copy(rows0.at[pl.ds(0, jlen)],
                                agg_hbm.at[c, pl.ds(obase, jlen)])

        @pl.when(cid == 0)
        def _():
            copy_out(0)

        @pl.when(cid == 1)
        def _():
            copy_out(1)

    return k(ean, cols, zeros_agg)


def _sc_scatter_cnt(cols, zeros1, ones1, epad, npad):
    per_w = epad // NW
    nchunks = per_w // CHUNK
    rows_per_tile = npad // NS
    spans = _row_spans(rows_per_tile)

    @functools.partial(
        pl.kernel,
        out_type=jax.ShapeDtypeStruct((NC * npad,), F32),
        mesh=_sc_mesh(),
        scratch_types=[
            pltpu.VMEM((CHUNK,), jnp.int32),
            pltpu.VMEM((CHUNK,), F32),
            pltpu.VMEM((CHUNK,), F32),
            pltpu.VMEM_SHARED((npad,), F32),
        ],
    )
    def k(col_hbm, z_hbm, ones_hbm, cnt_hbm, idx_v, stage_v, ones_v, sh_cnt):
        cid = lax.axis_index("c")
        sid = lax.axis_index("s")
        wid = sid * NC + cid

        pltpu.sync_copy(z_hbm, stage_v)
        for joff, jlen in spans:
            zbase = sid * rows_per_tile + joff
            pltpu.sync_copy(stage_v.at[pl.ds(0, jlen)],
                            sh_cnt.at[pl.ds(zbase, jlen)])
        pltpu.sync_copy(ones_hbm, ones_v)
        plsc.subcore_barrier()

        base_w = wid * per_w

        def body(i, carry):
            base = base_w + i * CHUNK
            pltpu.sync_copy(col_hbm.at[pl.ds(base, CHUNK)], idx_v)
            pltpu.sync_copy(ones_v, sh_cnt.at[idx_v], add=True)
            return carry

        lax.fori_loop(0, nchunks, body, 0)
        plsc.subcore_barrier()

        def copy_out(c):
            for joff, jlen in spans:
                obase = sid * rows_per_tile + joff
                pltpu.sync_copy(sh_cnt.at[pl.ds(obase, jlen)],
                                stage_v.at[pl.ds(0, jlen)])
                pltpu.sync_copy(stage_v.at[pl.ds(0, jlen)],
                                cnt_hbm.at[pl.ds(c * npad + obase, jlen)])

        @pl.when(cid == 0)
        def _():
            copy_out(0)

        @pl.when(cid == 1)
        def _():
            copy_out(1)

    return k(cols, zeros1, ones1)


def _tc_node(x, agg_p, c0, c1, w1x, w1a, b1, g, b, w2, b2):
    n, d = x.shape
    h = w1x.shape[1]
    cw = c0.shape[1]
    blk = 2000

    def body(x_ref, a_ref, c0_ref, c1_ref, w1x_ref, w1a_ref,
             b1_ref, g_ref, b_ref, w2_ref, b2_ref, out_ref):
        cnt = c0_ref[...][:, :1] + c1_ref[...][:, :1]
        inv = 1.0 / jnp.maximum(cnt, 1.0)
        agg = (a_ref[0] + a_ref[1]) * inv
        gg = (jnp.dot(x_ref[...], w1x_ref[...], preferred_element_type=F32)
              + jnp.dot(agg, w1a_ref[...], preferred_element_type=F32)
              + b1_ref[...])
        hh_ = jnp.maximum(gg, 0.0)
        m = jnp.mean(hh_, axis=-1, keepdims=True)
        c = hh_ - m
        v = jnp.mean(c * c, axis=-1, keepdims=True)
        hn = c * lax.rsqrt(v + LN_EPS) * g_ref[...] + b_ref[...]
        out_ref[...] = (jnp.dot(hn, w2_ref[...], preferred_element_type=F32)
                        + b2_ref[...])

    return pl.pallas_call(
        body,
        grid=(n // blk,),
        in_specs=[
            pl.BlockSpec((blk, d), lambda i: (i, 0)),
            pl.BlockSpec((NC, blk, h), lambda i: (0, i, 0)),
            pl.BlockSpec((blk, cw), lambda i: (i, 0)),
            pl.BlockSpec((blk, cw), lambda i: (i, 0)),
            pl.BlockSpec((d, h), lambda i: (0, 0)),
            pl.BlockSpec((h, h), lambda i: (0, 0)),
            pl.BlockSpec((1, h), lambda i: (0, 0)),
            pl.BlockSpec((1, h), lambda i: (0, 0)),
            pl.BlockSpec((1, h), lambda i: (0, 0)),
            pl.BlockSpec((h, h), lambda i: (0, 0)),
            pl.BlockSpec((1, h), lambda i: (0, 0)),
        ],
        out_specs=pl.BlockSpec((blk, h), lambda i: (i, 0)),
        out_shape=jax.ShapeDtypeStruct((n, h), F32),
    )(x, agg_p, c0, c1, w1x, w1a, b1, g, b, w2, b2)


def kernel(x, edge_index, edge_attr, e_W1, e_b1, e_ln_g, e_ln_b, e_W2, e_b2,
           n_W1, n_b1, n_ln_g, n_ln_b, n_W2, n_b2):
    n, d = x.shape
    e, ed = edge_attr.shape
    h = e_W2.shape[1]

    per_w_chunks = -(-e // (NW * CHUNK))
    per_w_chunks += per_w_chunks % 2
    epad = NW * CHUNK * per_w_chunks
    pad = epad - e

    row = edge_index[0]
    col = edge_index[1]
    pad_g = jnp.arange(pad, dtype=jnp.int32) % jnp.int32(128)
    rowg = jnp.concatenate([row, pad_g])
    colg = jnp.concatenate([col, pad_g])
    pad_s = jnp.int32(n) + jnp.arange(pad, dtype=jnp.int32) % jnp.int32(16)
    cols = jnp.concatenate([col, pad_s])
    ea_pad = jnp.concatenate([edge_attr, jnp.zeros((pad, ed), F32)], axis=0)

    w1s = e_W1[:d]
    w1d = e_W1[d:2 * d]
    w1e = e_W1[2 * d:]

    npad = -(-(n + 16) // 128) * 128

    xs, xd = _tc_precompute(x, w1s, w1d)
    gs, gd = _sc_gather(xs, xd, rowg, colg, epad)
    cnt1d = _sc_scatter_cnt(cols, jnp.zeros((CHUNK,), F32),
                            jnp.ones((CHUNK,), F32), epad, npad)
    ean_pad = _tc_edge(gs, gd, ea_pad, w1e,
                       e_b1.reshape(1, -1), e_ln_g.reshape(1, -1),
                       e_ln_b.reshape(1, -1), e_W2, e_b2.reshape(1, -1))
    edge_attr_new = ean_pad[:e]

    agg_p = _sc_scatter_agg(ean_pad, cols, jnp.zeros((CHUNK, h), F32), npad)
    c0 = cnt1d[:npad].reshape(npad, 1)
    c1 = cnt1d[npad:].reshape(npad, 1)

    x_new = _tc_node(x, agg_p, c0, c1,
                     n_W1[:d], n_W1[d:],
                     n_b1.reshape(1, -1), n_ln_g.reshape(1, -1),
                     n_ln_b.reshape(1, -1), n_W2, n_b2.reshape(1, -1))
    return (x_new, edge_attr_new)
